# baseline (speedup 1.0000x reference)
import functools

import jax
import jax.numpy as jnp
from jax import lax
from jax.experimental import pallas as pl
from jax.experimental.pallas import tpu as pltpu
from jax.experimental.pallas import tpu_sc as plsc

NC = 2
NS = 16
NW = NC * NS

_SCAN = 1600
_CAP = 4096
_GB = 128
_THRESH = 2048
_NEG = float("-inf")


def _sc_mesh():
    return plsc.VectorSubcoreMesh(
        core_axis_name="c", subcore_axis_name="s", num_cores=NC, num_subcores=NS
    )


def _make_scatter(E, H, NPAD):
    R = NPAD // NW
    n_chunks = E // _SCAN
    n_pairs = n_chunks // 2
    n_grp = _SCAN // 16
    hi = H // 16
    f_pairs = _CAP // _GB // 2

    @functools.partial(
        pl.kernel,
        mesh=_sc_mesh(),
        compiler_params=pltpu.CompilerParams(needs_layout_passes=False),
        out_type=(
            jax.ShapeDtypeStruct((NPAD * H,), jnp.float32),
            jax.ShapeDtypeStruct((NPAD * H,), jnp.float32),
        ),
        scratch_types=[
            pltpu.VMEM((_SCAN,), jnp.int32),
            pltpu.VMEM((_SCAN,), jnp.int32),
            pltpu.VMEM((_CAP + 16,), jnp.int32),
            pltpu.VMEM((_CAP + _GB + 16,), jnp.int32),
            pltpu.VMEM((_GB, H), jnp.float32),
            pltpu.VMEM((_GB, H), jnp.float32),
            pltpu.VMEM(((R + 1) * H,), jnp.float32),
            pltpu.VMEM(((R + 1) * H,), jnp.float32),
            pltpu.SemaphoreType.DMA,
            pltpu.SemaphoreType.DMA,
            pltpu.SemaphoreType.DMA,
            pltpu.SemaphoreType.DMA,
        ],
    )
    def scatter_k(ea_hbm, dst_hbm, msum_hbm, mmax_hbm, dstb0, dstb1, eidb,
                  ldstb, stage0, stage1, accs, accm, sd0, sd1, sg0, sg1):
        wid = lax.axis_index("s") * NC + lax.axis_index("c")
        base = wid * R
        iota = lax.iota(jnp.int32, 16)
        zf = jnp.zeros((16,), jnp.float32)
        ninf = jnp.full((16,), _NEG, jnp.float32)
        zi = jnp.zeros((16,), jnp.int32)
        rsplat = jnp.full((16,), R, jnp.int32)

        def init_acc(i, _):
            accs[pl.ds(i * 16, 16)] = zf
            accm[pl.ds(i * 16, 16)] = ninf
            return 0

        lax.fori_loop(0, (R + 1) * H // 16, init_acc, 0)

        def init_eid(i, _):
            eidb[pl.ds(i * 16, 16)] = zi
            return 0

        lax.fori_loop(0, (_CAP + 16) // 16, init_eid, 0)

        def dst_issue(c, buf, sem):
            pltpu.async_copy(dst_hbm.at[pl.ds(c * _SCAN, _SCAN)], buf, sem)

        def dst_wait(buf, sem):
            pltpu.make_async_copy(
                dst_hbm.at[pl.ds(0, _SCAN)], buf, sem
            ).wait()

        def g_issue(b, stg, sem):
            pltpu.async_copy(ea_hbm.at[eidb.at[pl.ds(b * _GB, _GB)]], stg, sem)

        def g_wait(stg, sem):
            pltpu.make_async_copy(
                ea_hbm.at[eidb.at[pl.ds(0, _GB)]], stg, sem
            ).wait()

        def accum_blk(b, stg):
            def grp4(g4, _):
                pos = b * _GB + g4 * 4
                dvec = ldstb[pl.ds(pos, 16)]
                offv = dvec * H
                for l in range(4):
                    o = offv[l]
                    i_row = g4 * 4 + l
                    rvs = [stg[i_row, pl.ds(16 * j, 16)] for j in range(hi)]
                    for j in range(hi):
                        plsc.addupdate(accs.at[pl.ds(o + 16 * j, 16)], rvs[j])
                    mxs = [accm[pl.ds(o + 16 * j, 16)] for j in range(hi)]
                    for j in range(hi):
                        accm[pl.ds(o + 16 * j, 16)] = jnp.maximum(mxs[j], rvs[j])
                return 0

            lax.fori_loop(0, _GB // 4, grp4, 0)

        def flush(ptr):
            for t in range(_GB // 16):
                ldstb[pl.ds(ptr + 16 * t, 16)] = rsplat
            g_issue(0, stage0, sg0)

            def fpair(q, _):
                b0 = 2 * q
                b1 = b0 + 1

                @pl.when(b0 * _GB < ptr)
                def _():
                    g_wait(stage0, sg0)

                    @pl.when(b1 * _GB < ptr)
                    def _():
                        g_issue(b1, stage1, sg1)

                    accum_blk(b0, stage0)

                @pl.when(b1 * _GB < ptr)
                def _():
                    g_wait(stage1, sg1)

                    @pl.when((b1 + 1) * _GB < ptr)
                    def _():
                        g_issue(b1 + 1, stage0, sg0)

                    accum_blk(b1, stage1)

                return 0

            lax.fori_loop(0, f_pairs, fpair, 0)

        def scan_groups(buf, c, ptr):
            def grp(g, p):
                for u in range(2):
                    dv = buf[pl.ds(g * 32 + u * 16, 16)]
                    m = (dv >= base) & (dv < base + R)
                    cnt = jnp.sum(m.astype(jnp.int32))
                    eidv = c * _SCAN + g * 32 + u * 16 + iota
                    plsc.store_compressed(eidb.at[pl.ds(p, 16)], eidv, mask=m)
                    plsc.store_compressed(ldstb.at[pl.ds(p, 16)], dv - base, mask=m)
                    p = p + cnt
                return p

            return lax.fori_loop(0, n_grp // 2, grp, ptr)

        def maybe_flush(ptr):
            @pl.when(ptr >= _THRESH)
            def _():
                flush(ptr)

            return jnp.where(ptr >= _THRESH, 0, ptr)

        dst_issue(0, dstb0, sd0)

        def spair(q, ptr):
            c0 = 2 * q
            c1 = c0 + 1
            dst_wait(dstb0, sd0)
            dst_issue(c1, dstb1, sd1)
            ptr = scan_groups(dstb0, c0, ptr)
            ptr = maybe_flush(ptr)
            dst_wait(dstb1, sd1)

            @pl.when(q < n_pairs - 1)
            def _():
                dst_issue(c0 + 2, dstb0, sd0)

            ptr = scan_groups(dstb1, c1, ptr)
            return maybe_flush(ptr)

        ptr = lax.fori_loop(0, n_pairs, spair, jnp.int32(0))

        @pl.when(ptr > 0)
        def _():
            flush(ptr)

        pltpu.sync_copy(accs.at[pl.ds(0, R * H)],
                        msum_hbm.at[pl.ds(base * H, R * H)])
        pltpu.sync_copy(accm.at[pl.ds(0, R * H)],
                        mmax_hbm.at[pl.ds(base * H, R * H)])

    return scatter_k


def _make_gather(N, E, H, base, Eh):
    EW = Eh // NW
    C = 200
    n_chunks = EW // C
    n_pairs = n_chunks // 2
    has_tail = (n_chunks % 2) == 1
    hi = H // 16

    @functools.partial(
        pl.kernel,
        mesh=_sc_mesh(),
        compiler_params=pltpu.CompilerParams(needs_layout_passes=False),
        out_type=jax.ShapeDtypeStruct((Eh, H), jnp.float32),
        scratch_types=[
            pltpu.VMEM((C,), jnp.int32),
            pltpu.VMEM((C,), jnp.int32),
            pltpu.VMEM((C,), jnp.int32),
            pltpu.VMEM((C,), jnp.int32),
            pltpu.VMEM((C, H), jnp.float32),
            pltpu.VMEM((C, H), jnp.float32),
            pltpu.VMEM((C, H), jnp.float32),
            pltpu.VMEM((C, H), jnp.float32),
            pltpu.SemaphoreType.DMA,
            pltpu.SemaphoreType.DMA,
            pltpu.SemaphoreType.DMA,
            pltpu.SemaphoreType.DMA,
            pltpu.SemaphoreType.DMA,
            pltpu.SemaphoreType.DMA,
        ],
    )
    def gather_k(u_hbm, v_hbm, src_hbm, dst_hbm, g_hbm,
                 sidx0, sidx1, didx0, didx1, bufu0, bufu1, bufv0, bufv1,
                 sgu0, sgu1, sgv0, sgv1, sw0, sw1):
        wid = lax.axis_index("s") * NC + lax.axis_index("c")
        woff = base + wid * EW
        goff = wid * EW
        sidx = (sidx0, sidx1)
        didx = (didx0, didx1)
        bufu = (bufu0, bufu1)
        bufv = (bufv0, bufv1)

        def idx_load(c, p):
            pltpu.sync_copy(src_hbm.at[pl.ds(woff + c * C, C)], sidx[p])
            pltpu.sync_copy(dst_hbm.at[pl.ds(woff + c * C, C)], didx[p])

        def g_issue(p, su, sv):
            pltpu.async_copy(u_hbm.at[sidx[p]], bufu[p], su)
            pltpu.async_copy(v_hbm.at[didx[p]], bufv[p], sv)

        def g_wait(p, su, sv):
            pltpu.make_async_copy(u_hbm.at[sidx[p]], bufu[p], su).wait()
            pltpu.make_async_copy(v_hbm.at[didx[p]], bufv[p], sv).wait()

        def w_issue(c, p, sw):
            pltpu.async_copy(bufu[p], g_hbm.at[pl.ds(goff + c * C, C)], sw)

        def w_wait(p, sw):
            pltpu.make_async_copy(
                bufu[p], g_hbm.at[pl.ds(0, C)], sw
            ).wait()

        def addv(pu, pv):
            def ab(i, _):
                vs = [pv[i, pl.ds(16 * j, 16)] for j in range(hi)]
                for j in range(hi):
                    plsc.addupdate(pu.at[i, pl.ds(16 * j, 16)], vs[j])
                return 0

            lax.fori_loop(0, C, ab, 0)

        idx_load(0, 0)
        g_issue(0, sgu0, sgv0)

        def pair(q, _):
            c0 = 2 * q
            c1 = c0 + 1
            g_wait(0, sgu0, sgv0)
            idx_load(c0 + 1, 1)

            @pl.when(q > 0)
            def _():
                w_wait(1, sw1)

            g_issue(1, sgu1, sgv1)
            addv(bufu0, bufv0)
            w_issue(c0, 0, sw0)
            g_wait(1, sgu1, sgv1)

            def prep_next():
                idx_load(c1 + 1, 0)
                w_wait(0, sw0)
                g_issue(0, sgu0, sgv0)

            if has_tail:
                prep_next()
            else:
                @pl.when(q < n_pairs - 1)
                def _():
                    prep_next()

            addv(bufu1, bufv1)
            w_issue(c1, 1, sw1)
            return 0

        lax.fori_loop(0, n_pairs, pair, 0)
        if has_tail:
            cT = n_chunks - 1
            g_wait(0, sgu0, sgv0)
            addv(bufu0, bufv0)
            w_issue(cT, 0, sw0)
        w_wait(0, sw0)
        w_wait(1, sw1)

    return gather_k


def _node_body(x_r, mm_r, ms_r, wx_r, wmx_r, wms_r, b1_r, g1_r, be1_r,
               w2_r, b2_r, weu_r, wev_r, xn_r, u_r, v_r):
    x = x_r[...]
    mm = mm_r[...]
    mm = jnp.where(mm == _NEG, 0.0, mm)
    h = jnp.dot(x, wx_r[...], preferred_element_type=jnp.float32)
    h += jnp.dot(mm, wmx_r[...], preferred_element_type=jnp.float32)
    h += jnp.dot(ms_r[...], wms_r[...], preferred_element_type=jnp.float32)
    h += b1_r[...]
    mu = jnp.mean(h, axis=-1, keepdims=True)
    var = jnp.mean((h - mu) ** 2, axis=-1, keepdims=True)
    hn = (h - mu) * lax.rsqrt(var + 1e-5) * g1_r[...] + be1_r[...]
    hr = jnp.maximum(hn, 0.0)
    xn = x + jnp.dot(hr, w2_r[...], preferred_element_type=jnp.float32) + b2_r[...]
    xn_r[...] = xn
    u_r[...] = jnp.dot(xn, weu_r[...], preferred_element_type=jnp.float32)
    v_r[...] = jnp.dot(xn, wev_r[...], preferred_element_type=jnp.float32)


def _edge_body(ea_r, g_r, wc_r, b1_r, g1e_r, be1e_r, w2_r, b2_r, out_r):
    _edge_common(ea_r, g_r, wc_r, b1_r, g1e_r, be1e_r, w2_r, b2_r, out_r)


def _edge_body_alias(ea_r, g_r, wc_r, b1_r, g1e_r, be1e_r, w2_r, b2_r,
                     prev_r, out_r):
    _edge_common(ea_r, g_r, wc_r, b1_r, g1e_r, be1e_r, w2_r, b2_r, out_r)


def _edge_common(ea_r, g_r, wc_r, b1_r, g1e_r, be1e_r, w2_r, b2_r, out_r):
    ea = ea_r[...]
    h = jnp.dot(ea, wc_r[...], preferred_element_type=jnp.float32)
    h += g_r[...] + b1_r[...]
    mu = jnp.mean(h, axis=-1, keepdims=True)
    var = jnp.mean((h - mu) ** 2, axis=-1, keepdims=True)
    hn = (h - mu) * lax.rsqrt(var + 1e-5) * g1e_r[...] + be1e_r[...]
    hr = jnp.maximum(hn, 0.0)
    out_r[...] = ea + jnp.dot(hr, w2_r[...], preferred_element_type=jnp.float32) + b2_r[...]


def kernel(x, edge_attr, edge_index, W1n, b1n, g1n, be1n, W2n, b2n,
           W1e, b1e, g1e, be1e, W2e, b2e):
    N, H = x.shape
    E = edge_attr.shape[0]
    NPAD = -(-N // (NW * 8)) * (NW * 8)

    src = edge_index[0]
    dst = edge_index[1]

    msum_f, mmax_f = _make_scatter(E, H, NPAD)(edge_attr, dst)
    msum = msum_f.reshape(NPAD, H)[:N]
    mmax = mmax_f.reshape(NPAD, H)[:N]

    row = lambda i: (i, 0)
    fixed = lambda i: (0, 0)
    BN = 1000
    w_spec = pl.BlockSpec((H, H), fixed)
    b_spec = pl.BlockSpec((1, H), fixed)
    n_spec = pl.BlockSpec((BN, H), row)
    x_new, U, V = pl.pallas_call(
        _node_body,
        grid=(N // BN,),
        in_specs=[n_spec, n_spec, n_spec, w_spec, w_spec, w_spec, b_spec,
                  b_spec, b_spec, w_spec, b_spec, w_spec, w_spec],
        out_specs=[n_spec, n_spec, n_spec],
        out_shape=[jax.ShapeDtypeStruct((N, H), jnp.float32)] * 3,
    )(
        x, mmax, msum,
        W1n[:H], W1n[H:2 * H], W1n[2 * H:],
        b1n.reshape(1, H), g1n.reshape(1, H), be1n.reshape(1, H),
        W2n, b2n.reshape(1, H),
        W1e[:H], W1e[H:2 * H],
    )

    Eh = E // 2
    BE = 2000
    nbh = Eh // BE
    e_spec = pl.BlockSpec((BE, H), row)
    wc = W1e[2 * H:]
    ew = (b1e.reshape(1, H), g1e.reshape(1, H), be1e.reshape(1, H),
          W2e, b2e.reshape(1, H))
    ew_specs = [b_spec, b_spec, b_spec, w_spec, b_spec]

    G0 = _make_gather(N, E, H, 0, Eh)(U, V, src, dst)
    G1 = _make_gather(N, E, H, Eh, Eh)(U, V, src, dst)

    e_half = pl.pallas_call(
        _edge_body,
        grid=(nbh,),
        in_specs=[e_spec, e_spec, w_spec] + ew_specs,
        out_specs=e_spec,
        out_shape=jax.ShapeDtypeStruct((E, H), jnp.float32),
    )(edge_attr, G0, wc, *ew)

    shift = lambda i: (i + nbh, 0)
    e_new = pl.pallas_call(
        _edge_body_alias,
        grid=(nbh,),
        in_specs=[pl.BlockSpec((BE, H), shift), e_spec, w_spec] + ew_specs
        + [pl.BlockSpec(memory_space=pl.ANY)],
        out_specs=pl.BlockSpec((BE, H), shift),
        out_shape=jax.ShapeDtypeStruct((E, H), jnp.float32),
        input_output_aliases={8: 0},
    )(edge_attr, G1, wc, *ew, e_half)

    return (x_new, e_new)


# speedup vs baseline: 1.0790x; 1.0790x over previous
import functools

import jax
import jax.numpy as jnp
from jax import lax
from jax.experimental import pallas as pl
from jax.experimental.pallas import tpu as pltpu
from jax.experimental.pallas import tpu_sc as plsc

NC = 2
NS = 16
NW = NC * NS

_SCAN = 2000
_CAP = 4096
_GB = 128
_THRESH = 2048
_NEG = float("-inf")


def _sc_mesh():
    return plsc.VectorSubcoreMesh(
        core_axis_name="c", subcore_axis_name="s", num_cores=NC, num_subcores=NS
    )


def _make_scatter(E, H, NPAD):
    R = NPAD // NW
    n_chunks = E // _SCAN
    n_pairs = n_chunks // 2
    n_grp = _SCAN // 16
    hi = H // 16
    f_pairs = _CAP // _GB // 2

    @functools.partial(
        pl.kernel,
        mesh=_sc_mesh(),
        compiler_params=pltpu.CompilerParams(needs_layout_passes=False),
        out_type=(
            jax.ShapeDtypeStruct((NPAD * H,), jnp.float32),
            jax.ShapeDtypeStruct((NPAD * H,), jnp.float32),
        ),
        scratch_types=[
            pltpu.VMEM((_SCAN,), jnp.int32),
            pltpu.VMEM((_SCAN,), jnp.int32),
            pltpu.VMEM((_CAP + 16,), jnp.int32),
            pltpu.VMEM((_CAP + _GB + 16,), jnp.int32),
            pltpu.VMEM((_GB, H), jnp.float32),
            pltpu.VMEM((_GB, H), jnp.float32),
            pltpu.VMEM(((R + 1) * H,), jnp.float32),
            pltpu.VMEM(((R + 1) * H,), jnp.float32),
            pltpu.SemaphoreType.DMA,
            pltpu.SemaphoreType.DMA,
            pltpu.SemaphoreType.DMA,
            pltpu.SemaphoreType.DMA,
        ],
    )
    def scatter_k(ea_hbm, dst_hbm, msum_hbm, mmax_hbm, dstb0, dstb1, eidb,
                  ldstb, stage0, stage1, accs, accm, sd0, sd1, sg0, sg1):
        wid = lax.axis_index("s") * NC + lax.axis_index("c")
        base = wid * R
        iota = lax.iota(jnp.int32, 16)
        zf = jnp.zeros((16,), jnp.float32)
        ninf = jnp.full((16,), _NEG, jnp.float32)
        zi = jnp.zeros((16,), jnp.int32)
        rsplat = jnp.full((16,), R, jnp.int32)

        def init_acc(i, _):
            accs[pl.ds(i * 16, 16)] = zf
            accm[pl.ds(i * 16, 16)] = ninf
            return 0

        lax.fori_loop(0, (R + 1) * H // 16, init_acc, 0)

        def init_eid(i, _):
            eidb[pl.ds(i * 16, 16)] = zi
            return 0

        lax.fori_loop(0, (_CAP + 16) // 16, init_eid, 0)

        def dst_issue(c, buf, sem):
            pltpu.async_copy(dst_hbm.at[pl.ds(c * _SCAN, _SCAN)], buf, sem)

        def dst_wait(buf, sem):
            pltpu.make_async_copy(
                dst_hbm.at[pl.ds(0, _SCAN)], buf, sem
            ).wait()

        def g_issue(b, stg, sem):
            pltpu.async_copy(ea_hbm.at[eidb.at[pl.ds(b * _GB, _GB)]], stg, sem)

        def g_wait(stg, sem):
            pltpu.make_async_copy(
                ea_hbm.at[eidb.at[pl.ds(0, _GB)]], stg, sem
            ).wait()

        def accum_blk(b, stg):
            def grp4(g4, _):
                pos = b * _GB + g4 * 4
                dvec = ldstb[pl.ds(pos, 16)]
                offv = dvec * H
                for l in range(4):
                    o = offv[l]
                    i_row = g4 * 4 + l
                    rvs = [stg[i_row, pl.ds(16 * j, 16)] for j in range(hi)]
                    for j in range(hi):
                        plsc.addupdate(accs.at[pl.ds(o + 16 * j, 16)], rvs[j])
                    mxs = [accm[pl.ds(o + 16 * j, 16)] for j in range(hi)]
                    for j in range(hi):
                        accm[pl.ds(o + 16 * j, 16)] = jnp.maximum(mxs[j], rvs[j])
                return 0

            lax.fori_loop(0, _GB // 4, grp4, 0)

        def flush(ptr):
            for t in range(_GB // 16):
                ldstb[pl.ds(ptr + 16 * t, 16)] = rsplat
            g_issue(0, stage0, sg0)

            def fpair(q, _):
                b0 = 2 * q
                b1 = b0 + 1

                @pl.when(b0 * _GB < ptr)
                def _():
                    g_wait(stage0, sg0)

                    @pl.when(b1 * _GB < ptr)
                    def _():
                        g_issue(b1, stage1, sg1)

                    accum_blk(b0, stage0)

                @pl.when(b1 * _GB < ptr)
                def _():
                    g_wait(stage1, sg1)

                    @pl.when((b1 + 1) * _GB < ptr)
                    def _():
                        g_issue(b1 + 1, stage0, sg0)

                    accum_blk(b1, stage1)

                return 0

            lax.fori_loop(0, f_pairs, fpair, 0)

        def scan_groups(buf, c, ptr):
            def grp(g, p):
                dv = buf[pl.ds(g * 16, 16)]
                m = (dv >= base) & (dv < base + R)
                cnt = jnp.sum(m.astype(jnp.int32))
                eidv = c * _SCAN + g * 16 + iota
                plsc.store_compressed(eidb.at[pl.ds(p, 16)], eidv, mask=m)
                plsc.store_compressed(ldstb.at[pl.ds(p, 16)], dv - base, mask=m)
                return p + cnt

            return lax.fori_loop(0, n_grp, grp, ptr)

        def maybe_flush(ptr):
            @pl.when(ptr >= _THRESH)
            def _():
                flush(ptr)

            return jnp.where(ptr >= _THRESH, 0, ptr)

        dst_issue(0, dstb0, sd0)

        def spair(q, ptr):
            c0 = 2 * q
            c1 = c0 + 1
            dst_wait(dstb0, sd0)
            dst_issue(c1, dstb1, sd1)
            ptr = scan_groups(dstb0, c0, ptr)
            ptr = maybe_flush(ptr)
            dst_wait(dstb1, sd1)

            @pl.when(q < n_pairs - 1)
            def _():
                dst_issue(c0 + 2, dstb0, sd0)

            ptr = scan_groups(dstb1, c1, ptr)
            return maybe_flush(ptr)

        ptr = lax.fori_loop(0, n_pairs, spair, jnp.int32(0))

        @pl.when(ptr > 0)
        def _():
            flush(ptr)

        pltpu.sync_copy(accs.at[pl.ds(0, R * H)],
                        msum_hbm.at[pl.ds(base * H, R * H)])
        pltpu.sync_copy(accm.at[pl.ds(0, R * H)],
                        mmax_hbm.at[pl.ds(base * H, R * H)])

    return scatter_k


def _make_gather(N, E, H, base, Eh):
    EW = Eh // NW
    C = 200
    n_chunks = EW // C
    n_pairs = n_chunks // 2
    has_tail = (n_chunks % 2) == 1
    hi = H // 16

    @functools.partial(
        pl.kernel,
        mesh=_sc_mesh(),
        compiler_params=pltpu.CompilerParams(needs_layout_passes=False),
        out_type=jax.ShapeDtypeStruct((Eh, H), jnp.float32),
        scratch_types=[
            pltpu.VMEM((C,), jnp.int32),
            pltpu.VMEM((C,), jnp.int32),
            pltpu.VMEM((C,), jnp.int32),
            pltpu.VMEM((C,), jnp.int32),
            pltpu.VMEM((C, H), jnp.float32),
            pltpu.VMEM((C, H), jnp.float32),
            pltpu.VMEM((C, H), jnp.float32),
            pltpu.VMEM((C, H), jnp.float32),
            pltpu.SemaphoreType.DMA,
            pltpu.SemaphoreType.DMA,
            pltpu.SemaphoreType.DMA,
            pltpu.SemaphoreType.DMA,
            pltpu.SemaphoreType.DMA,
            pltpu.SemaphoreType.DMA,
        ],
    )
    def gather_k(u_hbm, v_hbm, src_hbm, dst_hbm, g_hbm,
                 sidx0, sidx1, didx0, didx1, bufu0, bufu1, bufv0, bufv1,
                 sgu0, sgu1, sgv0, sgv1, sw0, sw1):
        wid = lax.axis_index("s") * NC + lax.axis_index("c")
        woff = base + wid * EW
        goff = wid * EW
        sidx = (sidx0, sidx1)
        didx = (didx0, didx1)
        bufu = (bufu0, bufu1)
        bufv = (bufv0, bufv1)

        def idx_load(c, p):
            pltpu.sync_copy(src_hbm.at[pl.ds(woff + c * C, C)], sidx[p])
            pltpu.sync_copy(dst_hbm.at[pl.ds(woff + c * C, C)], didx[p])

        def g_issue(p, su, sv):
            pltpu.async_copy(u_hbm.at[sidx[p]], bufu[p], su)
            pltpu.async_copy(v_hbm.at[didx[p]], bufv[p], sv)

        def g_wait(p, su, sv):
            pltpu.make_async_copy(u_hbm.at[sidx[p]], bufu[p], su).wait()
            pltpu.make_async_copy(v_hbm.at[didx[p]], bufv[p], sv).wait()

        def w_issue(c, p, sw):
            pltpu.async_copy(bufu[p], g_hbm.at[pl.ds(goff + c * C, C)], sw)

        def w_wait(p, sw):
            pltpu.make_async_copy(
                bufu[p], g_hbm.at[pl.ds(0, C)], sw
            ).wait()

        def addv(pu, pv):
            def ab(i, _):
                vs = [pv[i, pl.ds(16 * j, 16)] for j in range(hi)]
                for j in range(hi):
                    plsc.addupdate(pu.at[i, pl.ds(16 * j, 16)], vs[j])
                return 0

            lax.fori_loop(0, C, ab, 0)

        idx_load(0, 0)
        g_issue(0, sgu0, sgv0)

        def pair(q, _):
            c0 = 2 * q
            c1 = c0 + 1
            g_wait(0, sgu0, sgv0)
            idx_load(c0 + 1, 1)

            @pl.when(q > 0)
            def _():
                w_wait(1, sw1)

            g_issue(1, sgu1, sgv1)
            addv(bufu0, bufv0)
            w_issue(c0, 0, sw0)
            g_wait(1, sgu1, sgv1)

            def prep_next():
                idx_load(c1 + 1, 0)
                w_wait(0, sw0)
                g_issue(0, sgu0, sgv0)

            if has_tail:
                prep_next()
            else:
                @pl.when(q < n_pairs - 1)
                def _():
                    prep_next()

            addv(bufu1, bufv1)
            w_issue(c1, 1, sw1)
            return 0

        lax.fori_loop(0, n_pairs, pair, 0)
        if has_tail:
            cT = n_chunks - 1
            g_wait(0, sgu0, sgv0)
            addv(bufu0, bufv0)
            w_issue(cT, 0, sw0)
        w_wait(0, sw0)
        w_wait(1, sw1)

    return gather_k


def _node_body(x_r, mm_r, ms_r, wx_r, wmx_r, wms_r, b1_r, g1_r, be1_r,
               w2_r, b2_r, weu_r, wev_r, xn_r, u_r, v_r):
    x = x_r[...]
    mm = mm_r[...]
    mm = jnp.where(mm == _NEG, 0.0, mm)
    h = jnp.dot(x, wx_r[...], preferred_element_type=jnp.float32)
    h += jnp.dot(mm, wmx_r[...], preferred_element_type=jnp.float32)
    h += jnp.dot(ms_r[...], wms_r[...], preferred_element_type=jnp.float32)
    h += b1_r[...]
    mu = jnp.mean(h, axis=-1, keepdims=True)
    var = jnp.mean((h - mu) ** 2, axis=-1, keepdims=True)
    hn = (h - mu) * lax.rsqrt(var + 1e-5) * g1_r[...] + be1_r[...]
    hr = jnp.maximum(hn, 0.0)
    xn = x + jnp.dot(hr, w2_r[...], preferred_element_type=jnp.float32) + b2_r[...]
    xn_r[...] = xn
    u_r[...] = jnp.dot(xn, weu_r[...], preferred_element_type=jnp.float32)
    v_r[...] = jnp.dot(xn, wev_r[...], preferred_element_type=jnp.float32)


def _edge_body(ea_r, g_r, wc_r, b1_r, g1e_r, be1e_r, w2_r, b2_r, out_r):
    _edge_common(ea_r, g_r, wc_r, b1_r, g1e_r, be1e_r, w2_r, b2_r, out_r)


def _edge_body_alias(ea_r, g_r, wc_r, b1_r, g1e_r, be1e_r, w2_r, b2_r,
                     prev_r, out_r):
    _edge_common(ea_r, g_r, wc_r, b1_r, g1e_r, be1e_r, w2_r, b2_r, out_r)


def _edge_common(ea_r, g_r, wc_r, b1_r, g1e_r, be1e_r, w2_r, b2_r, out_r):
    ea = ea_r[...]
    h = jnp.dot(ea, wc_r[...], preferred_element_type=jnp.float32)
    h += g_r[...] + b1_r[...]
    mu = jnp.mean(h, axis=-1, keepdims=True)
    var = jnp.mean((h - mu) ** 2, axis=-1, keepdims=True)
    hn = (h - mu) * lax.rsqrt(var + 1e-5) * g1e_r[...] + be1e_r[...]
    hr = jnp.maximum(hn, 0.0)
    out_r[...] = ea + jnp.dot(hr, w2_r[...], preferred_element_type=jnp.float32) + b2_r[...]


def kernel(x, edge_attr, edge_index, W1n, b1n, g1n, be1n, W2n, b2n,
           W1e, b1e, g1e, be1e, W2e, b2e):
    N, H = x.shape
    E = edge_attr.shape[0]
    NPAD = -(-N // (NW * 8)) * (NW * 8)

    src = edge_index[0]
    dst = edge_index[1]

    msum_f, mmax_f = _make_scatter(E, H, NPAD)(edge_attr, dst)
    msum = msum_f.reshape(NPAD, H)[:N]
    mmax = mmax_f.reshape(NPAD, H)[:N]

    row = lambda i: (i, 0)
    fixed = lambda i: (0, 0)
    BN = 1000
    w_spec = pl.BlockSpec((H, H), fixed)
    b_spec = pl.BlockSpec((1, H), fixed)
    n_spec = pl.BlockSpec((BN, H), row)
    x_new, U, V = pl.pallas_call(
        _node_body,
        grid=(N // BN,),
        in_specs=[n_spec, n_spec, n_spec, w_spec, w_spec, w_spec, b_spec,
                  b_spec, b_spec, w_spec, b_spec, w_spec, w_spec],
        out_specs=[n_spec, n_spec, n_spec],
        out_shape=[jax.ShapeDtypeStruct((N, H), jnp.float32)] * 3,
    )(
        x, mmax, msum,
        W1n[:H], W1n[H:2 * H], W1n[2 * H:],
        b1n.reshape(1, H), g1n.reshape(1, H), be1n.reshape(1, H),
        W2n, b2n.reshape(1, H),
        W1e[:H], W1e[H:2 * H],
    )

    Eh = E // 2
    BE = 2000
    nbh = Eh // BE
    e_spec = pl.BlockSpec((BE, H), row)
    wc = W1e[2 * H:]
    ew = (b1e.reshape(1, H), g1e.reshape(1, H), be1e.reshape(1, H),
          W2e, b2e.reshape(1, H))
    ew_specs = [b_spec, b_spec, b_spec, w_spec, b_spec]

    G0 = _make_gather(N, E, H, 0, Eh)(U, V, src, dst)
    G1 = _make_gather(N, E, H, Eh, Eh)(U, V, src, dst)

    e_half = pl.pallas_call(
        _edge_body,
        grid=(nbh,),
        in_specs=[e_spec, e_spec, w_spec] + ew_specs,
        out_specs=e_spec,
        out_shape=jax.ShapeDtypeStruct((E, H), jnp.float32),
    )(edge_attr, G0, wc, *ew)

    shift = lambda i: (i + nbh, 0)
    e_new = pl.pallas_call(
        _edge_body_alias,
        grid=(nbh,),
        in_specs=[pl.BlockSpec((BE, H), shift), e_spec, w_spec] + ew_specs
        + [pl.BlockSpec(memory_space=pl.ANY)],
        out_specs=pl.BlockSpec((BE, H), shift),
        out_shape=jax.ShapeDtypeStruct((E, H), jnp.float32),
        input_output_aliases={8: 0},
    )(edge_attr, G1, wc, *ew, e_half)

    return (x_new, e_new)
